# Initial kernel scaffold; baseline (speedup 1.0000x reference)
import functools

import jax
import jax.numpy as jnp
from jax import lax
from jax.experimental import pallas as pl
from jax.experimental.pallas import tpu as pltpu
from jax.experimental.pallas import tpu_sc as plsc

NC, NS = 2, 16
NW = NC * NS
CHUNK = 128


def _make_sc_agg(n_acc, n_chunks, d):
    rpt = n_acc // NS

    mesh = plsc.VectorSubcoreMesh(core_axis_name="c", subcore_axis_name="s")

    @functools.partial(
        pl.kernel,
        out_type=jax.ShapeDtypeStruct((NC, n_acc, d), jnp.float32),
        mesh=mesh,
        scratch_types=[
            pltpu.VMEM((n_chunks, CHUNK), jnp.int32),
            pltpu.VMEM((n_chunks, CHUNK), jnp.int32),
            pltpu.VMEM((CHUNK, d), jnp.float32),
            pltpu.VMEM((CHUNK, d), jnp.float32),
            pltpu.VMEM_SHARED((n_acc, d), jnp.float32),
        ],
    )
    def sc_agg(h_hbm, src_hbm, dst_hbm, out_hbm, src_v, dst_v, rows_v,
               zero_v, acc_sh):
        c = lax.axis_index("c")
        s = lax.axis_index("s")
        w = c * NS + s

        pltpu.sync_copy(src_hbm.at[w], src_v)
        pltpu.sync_copy(dst_hbm.at[w], dst_v)

        z16 = jnp.zeros((16,), jnp.float32)

        def zrow(r, carry):
            for c16 in range(d // 16):
                zero_v[r, pl.ds(c16 * 16, 16)] = z16
            return carry

        lax.fori_loop(0, CHUNK, zrow, 0)
        n_full = rpt // CHUNK
        for b in range(n_full):
            pltpu.sync_copy(
                zero_v, acc_sh.at[pl.ds(s * rpt + b * CHUNK, CHUNK)])
        rem = rpt - n_full * CHUNK
        if rem:
            pltpu.sync_copy(
                zero_v.at[pl.ds(0, rem)],
                acc_sh.at[pl.ds(s * rpt + n_full * CHUNK, rem)])
        plsc.subcore_barrier()

        def body(j, carry):
            pltpu.sync_copy(h_hbm.at[src_v.at[j]], rows_v)
            pltpu.sync_copy(rows_v, acc_sh.at[dst_v.at[j]], add=True)
            return carry

        lax.fori_loop(0, n_chunks, body, 0)
        plsc.subcore_barrier()

        pltpu.sync_copy(acc_sh.at[pl.ds(s * rpt, rpt)],
                        out_hbm.at[c].at[pl.ds(s * rpt, rpt)])

    return sc_agg


def _mlp_body(x_ref, p_ref, w1_ref, b1_ref, w2_ref, b2_ref, o_ref):
    n = x_ref.shape[0]
    m = x_ref[...] + p_ref[0, :n, :] + p_ref[1, :n, :]
    a = jnp.maximum(
        jnp.dot(m, w1_ref[...], preferred_element_type=jnp.float32)
        + b1_ref[...], 0.0)
    o_ref[...] = jnp.maximum(
        jnp.dot(a, w2_ref[...], preferred_element_type=jnp.float32)
        + b2_ref[...], 0.0)


def _mlp_pool_body(x_ref, p_ref, w1_ref, b1_ref, w2_ref, b2_ref, batch_ref,
                   o_ref):
    n = x_ref.shape[0]
    g = o_ref.shape[0]
    m = x_ref[...] + p_ref[0, :n, :] + p_ref[1, :n, :]
    a = jnp.maximum(
        jnp.dot(m, w1_ref[...], preferred_element_type=jnp.float32)
        + b1_ref[...], 0.0)
    h = jnp.maximum(
        jnp.dot(a, w2_ref[...], preferred_element_type=jnp.float32)
        + b2_ref[...], 0.0)
    gids = lax.broadcasted_iota(jnp.int32, (n, g), 1)
    oh = (batch_ref[...] == gids).astype(jnp.float32)
    sums = lax.dot_general(oh, h, (((0,), (0,)), ((), ())),
                           preferred_element_type=jnp.float32)
    counts = lax.dot_general(oh, jnp.ones((n, 1), jnp.float32),
                             (((0,), (0,)), ((), ())),
                             preferred_element_type=jnp.float32)
    o_ref[...] = sums / jnp.maximum(counts, 1.0)


def kernel(x, edge_index, batch, W1_0, b1_0, W2_0, b2_0, W1_1, b1_1, W2_1,
           b2_1, W1_2, b1_2, W2_2, b2_2):
    n_nodes, d = x.shape
    e = edge_index.shape[1]
    g = 64
    h_dim = W1_0.shape[1]

    n_acc = -(-(n_nodes + 1) // NS) * NS
    n_chunks = -(-e // (NW * CHUNK))
    if n_chunks % 2:
        n_chunks += 1
    e_pad = NW * n_chunks * CHUNK

    src = jnp.concatenate(
        [edge_index[0], jnp.zeros((e_pad - e,), jnp.int32)]
    ).reshape(NW, n_chunks, CHUNK)
    dst = jnp.concatenate(
        [edge_index[1], jnp.full((e_pad - e,), n_nodes, jnp.int32)]
    ).reshape(NW, n_chunks, CHUNK)

    sc_agg = _make_sc_agg(n_acc, n_chunks, d)

    mlp = pl.pallas_call(
        _mlp_body,
        out_shape=jax.ShapeDtypeStruct((n_nodes, h_dim), jnp.float32),
    )
    mlp_pool = pl.pallas_call(
        _mlp_pool_body,
        out_shape=jax.ShapeDtypeStruct((g, h_dim), jnp.float32),
    )

    params = [(W1_0, b1_0, W2_0, b2_0), (W1_1, b1_1, W2_1, b2_1),
              (W1_2, b1_2, W2_2, b2_2)]
    batch2d = batch.reshape(n_nodes, 1)

    h = x
    for i, (w1, b1, w2, b2) in enumerate(params):
        partials = sc_agg(h, src, dst)
        b1r = b1.reshape(1, h_dim)
        b2r = b2.reshape(1, h_dim)
        if i < 2:
            h = mlp(h, partials, w1, b1r, w2, b2r)
        else:
            h = mlp_pool(h, partials, w1, b1r, w2, b2r, batch2d)
    return h


# speedup vs baseline: 2.9142x; 2.9142x over previous
import functools

import jax
import jax.numpy as jnp
from jax import lax
from jax.experimental import pallas as pl
from jax.experimental.pallas import tpu as pltpu
from jax.experimental.pallas import tpu_sc as plsc

NC, NS = 2, 16
NW = NC * NS
CHUNK = 128


def _make_sc_agg(n_acc, n_chunks, d):
    rpt = n_acc // NS

    mesh = plsc.VectorSubcoreMesh(core_axis_name="c", subcore_axis_name="s")

    @functools.partial(
        pl.kernel,
        out_type=jax.ShapeDtypeStruct((NC, n_acc, d), jnp.float32),
        mesh=mesh,
        scratch_types=[
            pltpu.VMEM((n_chunks, CHUNK), jnp.int32),
            pltpu.VMEM((n_chunks, CHUNK), jnp.int32),
            pltpu.VMEM((CHUNK, d), jnp.float32),
            pltpu.VMEM_SHARED((n_acc, d), jnp.float32),
        ],
    )
    def sc_agg(h_hbm, src_hbm, dst_hbm, out_hbm, src_v, dst_v, rows_v,
               acc_sh):
        c = lax.axis_index("c")
        s = lax.axis_index("s")
        w = c * NS + s

        pltpu.sync_copy(src_hbm.at[w], src_v)
        pltpu.sync_copy(dst_hbm.at[w], dst_v)

        z16 = jnp.zeros((16,), jnp.float32)

        def zrow(r, carry):
            for c16 in range(d // 16):
                rows_v[r, pl.ds(c16 * 16, 16)] = z16
            return carry

        lax.fori_loop(0, CHUNK, zrow, 0)
        n_full = rpt // CHUNK
        for b in range(n_full):
            pltpu.sync_copy(
                rows_v, acc_sh.at[pl.ds(s * rpt + b * CHUNK, CHUNK)])
        rem = rpt - n_full * CHUNK
        if rem:
            pltpu.sync_copy(
                rows_v.at[pl.ds(0, rem)],
                acc_sh.at[pl.ds(s * rpt + n_full * CHUNK, rem)])
        plsc.subcore_barrier()

        def body(j, carry):
            pltpu.sync_copy(h_hbm.at[src_v.at[j]], rows_v)
            pltpu.sync_copy(rows_v, acc_sh.at[dst_v.at[j]], add=True)
            return carry

        lax.fori_loop(0, n_chunks, body, 0)
        plsc.subcore_barrier()

        pltpu.sync_copy(acc_sh.at[pl.ds(s * rpt, rpt)],
                        out_hbm.at[c].at[pl.ds(s * rpt, rpt)])

    return sc_agg


def _mlp_body(x_ref, p_ref, w1_ref, b1_ref, w2_ref, b2_ref, o_ref):
    n = x_ref.shape[0]
    m = x_ref[...] + p_ref[0, :n, :] + p_ref[1, :n, :]
    a = jnp.maximum(
        jnp.dot(m, w1_ref[...], preferred_element_type=jnp.float32)
        + b1_ref[...], 0.0)
    o_ref[...] = jnp.maximum(
        jnp.dot(a, w2_ref[...], preferred_element_type=jnp.float32)
        + b2_ref[...], 0.0)


def _mlp_pool_body(x_ref, p_ref, w1_ref, b1_ref, w2_ref, b2_ref, batch_ref,
                   o_ref):
    n = x_ref.shape[0]
    g = o_ref.shape[0]
    m = x_ref[...] + p_ref[0, :n, :] + p_ref[1, :n, :]
    a = jnp.maximum(
        jnp.dot(m, w1_ref[...], preferred_element_type=jnp.float32)
        + b1_ref[...], 0.0)
    h = jnp.maximum(
        jnp.dot(a, w2_ref[...], preferred_element_type=jnp.float32)
        + b2_ref[...], 0.0)
    gids = lax.broadcasted_iota(jnp.int32, (n, g), 1)
    oh = (batch_ref[...] == gids).astype(jnp.float32)
    sums = lax.dot_general(oh, h, (((0,), (0,)), ((), ())),
                           preferred_element_type=jnp.float32)
    counts = lax.dot_general(oh, jnp.ones((n, 1), jnp.float32),
                             (((0,), (0,)), ((), ())),
                             preferred_element_type=jnp.float32)
    o_ref[...] = sums / jnp.maximum(counts, 1.0)


def kernel(x, edge_index, batch, W1_0, b1_0, W2_0, b2_0, W1_1, b1_1, W2_1,
           b2_1, W1_2, b1_2, W2_2, b2_2):
    n_nodes, d = x.shape
    e = edge_index.shape[1]
    g = 64
    h_dim = W1_0.shape[1]

    n_acc = -(-(n_nodes + 1) // (8 * NS)) * (8 * NS)
    n_chunks = -(-e // (NW * CHUNK))
    if n_chunks % 2:
        n_chunks += 1
    e_pad = NW * n_chunks * CHUNK

    src = jnp.concatenate(
        [edge_index[0], jnp.zeros((e_pad - e,), jnp.int32)]
    ).reshape(NW, n_chunks, CHUNK)
    dst = jnp.concatenate(
        [edge_index[1], jnp.full((e_pad - e,), n_nodes, jnp.int32)]
    ).reshape(NW, n_chunks, CHUNK)

    sc_agg = _make_sc_agg(n_acc, n_chunks, d)

    mlp = pl.pallas_call(
        _mlp_body,
        out_shape=jax.ShapeDtypeStruct((n_nodes, h_dim), jnp.float32),
    )
    mlp_pool = pl.pallas_call(
        _mlp_pool_body,
        out_shape=jax.ShapeDtypeStruct((g, h_dim), jnp.float32),
    )

    params = [(W1_0, b1_0, W2_0, b2_0), (W1_1, b1_1, W2_1, b2_1),
              (W1_2, b1_2, W2_2, b2_2)]
    batch2d = batch.reshape(n_nodes, 1)

    h = x
    for i, (w1, b1, w2, b2) in enumerate(params):
        partials = sc_agg(h, src, dst)
        b1r = b1.reshape(1, h_dim)
        b2r = b2.reshape(1, h_dim)
        if i < 2:
            h = mlp(h, partials, w1, b1r, w2, b2r)
        else:
            h = mlp_pool(h, partials, w1, b1r, w2, b2r, batch2d)
    return h


# speedup vs baseline: 3.1530x; 1.0819x over previous
import functools

import jax
import jax.numpy as jnp
from jax import lax
from jax.experimental import pallas as pl
from jax.experimental.pallas import tpu as pltpu
from jax.experimental.pallas import tpu_sc as plsc

NC, NS = 2, 16
NW = NC * NS
CHUNK = 128


def _make_sc_agg(n_acc, n_chunks, d):
    rpt = n_acc // NS
    n_half = n_chunks // 2
    n_pairs = n_half // 2

    mesh = plsc.VectorSubcoreMesh(core_axis_name="c", subcore_axis_name="s")

    @functools.partial(
        pl.kernel,
        out_type=jax.ShapeDtypeStruct((NC, n_acc, d), jnp.float32),
        mesh=mesh,
        scratch_types=[
            pltpu.VMEM((n_half, CHUNK), jnp.int32),
            pltpu.VMEM((n_half, CHUNK), jnp.int32),
            pltpu.VMEM((CHUNK, d), jnp.float32),
            pltpu.VMEM((CHUNK, d), jnp.float32),
            pltpu.SemaphoreType.DMA,
            pltpu.SemaphoreType.DMA,
            pltpu.VMEM_SHARED((n_acc, d), jnp.float32),
        ],
    )
    def sc_agg(h_hbm, src_hbm, dst_hbm, out_hbm, src_v, dst_v, rows_a,
               rows_b, sem_a, sem_b, acc_sh):
        c = lax.axis_index("c")
        s = lax.axis_index("s")
        w = c * NS + s

        z16 = jnp.zeros((16,), jnp.float32)

        def zrow(r, carry):
            for c16 in range(d // 16):
                rows_a[r, pl.ds(c16 * 16, 16)] = z16
            return carry

        lax.fori_loop(0, CHUNK, zrow, 0)
        n_full = rpt // CHUNK
        for b in range(n_full):
            pltpu.sync_copy(
                rows_a, acc_sh.at[pl.ds(s * rpt + b * CHUNK, CHUNK)])
        rem = rpt - n_full * CHUNK
        if rem:
            pltpu.sync_copy(
                rows_a.at[pl.ds(0, rem)],
                acc_sh.at[pl.ds(s * rpt + n_full * CHUNK, rem)])
        plsc.subcore_barrier()

        for hf in range(2):
            pltpu.sync_copy(src_hbm.at[w].at[pl.ds(hf * n_half, n_half)],
                            src_v)
            pltpu.sync_copy(dst_hbm.at[w].at[pl.ds(hf * n_half, n_half)],
                            dst_v)
            pltpu.async_copy(h_hbm.at[src_v.at[0]], rows_a, sem_a)

            def body(p, carry):
                ja = 2 * p
                jb = 2 * p + 1
                pltpu.make_async_copy(
                    h_hbm.at[src_v.at[ja]], rows_a, sem_a).wait()
                pltpu.async_copy(h_hbm.at[src_v.at[jb]], rows_b, sem_b)
                pltpu.sync_copy(rows_a, acc_sh.at[dst_v.at[ja]], add=True)
                pltpu.make_async_copy(
                    h_hbm.at[src_v.at[jb]], rows_b, sem_b).wait()

                @pl.when(p < n_pairs - 1)
                def _():
                    pltpu.async_copy(
                        h_hbm.at[src_v.at[ja + 2]], rows_a, sem_a)

                pltpu.sync_copy(rows_b, acc_sh.at[dst_v.at[jb]], add=True)
                return carry

            lax.fori_loop(0, n_pairs, body, 0)
        plsc.subcore_barrier()

        pltpu.sync_copy(acc_sh.at[pl.ds(s * rpt, rpt)],
                        out_hbm.at[c].at[pl.ds(s * rpt, rpt)])

    return sc_agg


def _mlp_body(x_ref, p_ref, w1_ref, b1_ref, w2_ref, b2_ref, o_ref):
    n = x_ref.shape[0]
    m = x_ref[...] + p_ref[0, :n, :] + p_ref[1, :n, :]
    a = jnp.maximum(
        jnp.dot(m, w1_ref[...], preferred_element_type=jnp.float32)
        + b1_ref[...], 0.0)
    o_ref[...] = jnp.maximum(
        jnp.dot(a, w2_ref[...], preferred_element_type=jnp.float32)
        + b2_ref[...], 0.0)


def _mlp_pool_body(x_ref, p_ref, w1_ref, b1_ref, w2_ref, b2_ref, batch_ref,
                   o_ref):
    n = x_ref.shape[0]
    g = o_ref.shape[0]
    m = x_ref[...] + p_ref[0, :n, :] + p_ref[1, :n, :]
    a = jnp.maximum(
        jnp.dot(m, w1_ref[...], preferred_element_type=jnp.float32)
        + b1_ref[...], 0.0)
    h = jnp.maximum(
        jnp.dot(a, w2_ref[...], preferred_element_type=jnp.float32)
        + b2_ref[...], 0.0)
    gids = lax.broadcasted_iota(jnp.int32, (n, g), 1)
    oh = (batch_ref[...] == gids).astype(jnp.float32)
    sums = lax.dot_general(oh, h, (((0,), (0,)), ((), ())),
                           preferred_element_type=jnp.float32)
    counts = lax.dot_general(oh, jnp.ones((n, 1), jnp.float32),
                             (((0,), (0,)), ((), ())),
                             preferred_element_type=jnp.float32)
    o_ref[...] = sums / jnp.maximum(counts, 1.0)


def kernel(x, edge_index, batch, W1_0, b1_0, W2_0, b2_0, W1_1, b1_1, W2_1,
           b2_1, W1_2, b1_2, W2_2, b2_2):
    n_nodes, d = x.shape
    e = edge_index.shape[1]
    g = 64
    h_dim = W1_0.shape[1]

    n_acc = -(-(n_nodes + 1) // (8 * NS)) * (8 * NS)
    n_chunks = -(-e // (NW * CHUNK * 4)) * 4
    e_pad = NW * n_chunks * CHUNK

    src = jnp.concatenate(
        [edge_index[0], jnp.zeros((e_pad - e,), jnp.int32)]
    ).reshape(NW, n_chunks, CHUNK)
    dst = jnp.concatenate(
        [edge_index[1], jnp.full((e_pad - e,), n_nodes, jnp.int32)]
    ).reshape(NW, n_chunks, CHUNK)

    sc_agg = _make_sc_agg(n_acc, n_chunks, d)

    mlp = pl.pallas_call(
        _mlp_body,
        out_shape=jax.ShapeDtypeStruct((n_nodes, h_dim), jnp.float32),
    )
    mlp_pool = pl.pallas_call(
        _mlp_pool_body,
        out_shape=jax.ShapeDtypeStruct((g, h_dim), jnp.float32),
    )

    params = [(W1_0, b1_0, W2_0, b2_0), (W1_1, b1_1, W2_1, b2_1),
              (W1_2, b1_2, W2_2, b2_2)]
    batch2d = batch.reshape(n_nodes, 1)

    h = x
    for i, (w1, b1, w2, b2) in enumerate(params):
        partials = sc_agg(h, src, dst)
        b1r = b1.reshape(1, h_dim)
        b2r = b2.reshape(1, h_dim)
        if i < 2:
            h = mlp(h, partials, w1, b1r, w2, b2r)
        else:
            h = mlp_pool(h, partials, w1, b1r, w2, b2r, batch2d)
    return h


# speedup vs baseline: 3.4960x; 1.1088x over previous
import functools

import jax
import jax.numpy as jnp
from jax import lax
from jax.experimental import pallas as pl
from jax.experimental.pallas import tpu as pltpu
from jax.experimental.pallas import tpu_sc as plsc

NC, NS = 2, 16
NW = NC * NS
CHUNK = 128


def _make_sc_agg(n_acc, n_chunks, d):
    rpt = n_acc // NS
    n_half = n_chunks // 2
    n_pairs = n_half // 2

    mesh = plsc.VectorSubcoreMesh(core_axis_name="c", subcore_axis_name="s")

    @functools.partial(
        pl.kernel,
        out_type=jax.ShapeDtypeStruct((NC, n_acc, d), jnp.float32),
        mesh=mesh,
        scratch_types=[
            pltpu.VMEM((n_half, CHUNK), jnp.int32),
            pltpu.VMEM((n_half, CHUNK), jnp.int32),
            pltpu.VMEM((CHUNK, d), jnp.float32),
            pltpu.VMEM((CHUNK, d), jnp.float32),
            pltpu.SemaphoreType.DMA,
            pltpu.SemaphoreType.DMA,
            pltpu.VMEM_SHARED((n_acc, d), jnp.float32),
        ],
    )
    def sc_agg(h_hbm, src_hbm, dst_hbm, out_hbm, src_v, dst_v, rows_a,
               rows_b, sem_a, sem_b, acc_sh):
        c = lax.axis_index("c")
        s = lax.axis_index("s")
        w = c * NS + s

        z16 = jnp.zeros((16,), jnp.float32)

        def zrow(r, carry):
            for c16 in range(d // 16):
                rows_a[r, pl.ds(c16 * 16, 16)] = z16
            return carry

        lax.fori_loop(0, CHUNK, zrow, 0)
        n_full = rpt // CHUNK
        for b in range(n_full):
            pltpu.sync_copy(
                rows_a, acc_sh.at[pl.ds(s * rpt + b * CHUNK, CHUNK)])
        rem = rpt - n_full * CHUNK
        if rem:
            pltpu.sync_copy(
                rows_a.at[pl.ds(0, rem)],
                acc_sh.at[pl.ds(s * rpt + n_full * CHUNK, rem)])
        plsc.subcore_barrier()

        for hf in range(2):
            pltpu.sync_copy(src_hbm.at[w].at[pl.ds(hf * n_half, n_half)],
                            src_v)
            pltpu.sync_copy(dst_hbm.at[w].at[pl.ds(hf * n_half, n_half)],
                            dst_v)
            pltpu.async_copy(h_hbm.at[src_v.at[0]], rows_a, sem_a)

            def body(p, carry):
                ja = 2 * p
                jb = 2 * p + 1
                pltpu.make_async_copy(
                    h_hbm.at[src_v.at[ja]], rows_a, sem_a).wait()
                pltpu.async_copy(h_hbm.at[src_v.at[jb]], rows_b, sem_b)
                pltpu.sync_copy(rows_a, acc_sh.at[dst_v.at[ja]], add=True)
                pltpu.make_async_copy(
                    h_hbm.at[src_v.at[jb]], rows_b, sem_b).wait()

                @pl.when(p < n_pairs - 1)
                def _():
                    pltpu.async_copy(
                        h_hbm.at[src_v.at[ja + 2]], rows_a, sem_a)

                pltpu.sync_copy(rows_b, acc_sh.at[dst_v.at[jb]], add=True)
                return carry

            lax.fori_loop(0, n_pairs, body, 0)
        plsc.subcore_barrier()

        pltpu.sync_copy(acc_sh.at[pl.ds(s * rpt, rpt)],
                        out_hbm.at[c].at[pl.ds(s * rpt, rpt)])

    return sc_agg


def _mlp_body(x_ref, p_ref, w1_ref, b1_ref, w2_ref, b2_ref, o_ref):
    n = x_ref.shape[0]
    m = x_ref[...] + p_ref[0, :n, :] + p_ref[1, :n, :]
    a = jnp.maximum(
        jnp.dot(m, w1_ref[...], preferred_element_type=jnp.float32)
        + b1_ref[...], 0.0)
    o_ref[...] = jnp.maximum(
        jnp.dot(a, w2_ref[...], preferred_element_type=jnp.float32)
        + b2_ref[...], 0.0)


def _mlp_pool_body(x_ref, p_ref, w1_ref, b1_ref, w2_ref, b2_ref, batch_ref,
                   o_ref):
    n = x_ref.shape[0]
    g = o_ref.shape[0]
    m = x_ref[...] + p_ref[0, :n, :] + p_ref[1, :n, :]
    a = jnp.maximum(
        jnp.dot(m, w1_ref[...], preferred_element_type=jnp.float32)
        + b1_ref[...], 0.0)
    h = jnp.maximum(
        jnp.dot(a, w2_ref[...], preferred_element_type=jnp.float32)
        + b2_ref[...], 0.0)
    gids = lax.broadcasted_iota(jnp.int32, (n, g), 1)
    oh = (batch_ref[...] == gids).astype(jnp.float32)
    sums = lax.dot_general(oh, h, (((0,), (0,)), ((), ())),
                           preferred_element_type=jnp.float32)
    counts = lax.dot_general(oh, jnp.ones((n, 1), jnp.float32),
                             (((0,), (0,)), ((), ())),
                             preferred_element_type=jnp.float32)
    o_ref[...] = sums / jnp.maximum(counts, 1.0)


def kernel(x, edge_index, batch, W1_0, b1_0, W2_0, b2_0, W1_1, b1_1, W2_1,
           b2_1, W1_2, b1_2, W2_2, b2_2):
    n_nodes, d = x.shape
    e = edge_index.shape[1]
    g = 64
    h_dim = W1_0.shape[1]

    n_acc = -(-(n_nodes + 1) // (8 * NS)) * (8 * NS)
    n_chunks = -(-e // (NW * CHUNK * 4)) * 4
    e_pad = NW * n_chunks * CHUNK

    pad = e_pad - e
    spare = n_acc - n_nodes
    src_flat = jnp.concatenate(
        [edge_index[0], jnp.zeros((pad,), jnp.int32)])
    dst_flat = jnp.concatenate(
        [edge_index[1],
         n_nodes + (jnp.arange(pad, dtype=jnp.int32) % spare)])
    src = src_flat.reshape(n_chunks * CHUNK, NW).T.reshape(
        NW, n_chunks, CHUNK)
    dst = dst_flat.reshape(n_chunks * CHUNK, NW).T.reshape(
        NW, n_chunks, CHUNK)

    sc_agg = _make_sc_agg(n_acc, n_chunks, d)

    mlp = pl.pallas_call(
        _mlp_body,
        out_shape=jax.ShapeDtypeStruct((n_nodes, h_dim), jnp.float32),
    )
    mlp_pool = pl.pallas_call(
        _mlp_pool_body,
        out_shape=jax.ShapeDtypeStruct((g, h_dim), jnp.float32),
    )

    params = [(W1_0, b1_0, W2_0, b2_0), (W1_1, b1_1, W2_1, b2_1),
              (W1_2, b1_2, W2_2, b2_2)]
    batch2d = batch.reshape(n_nodes, 1)

    h = x
    for i, (w1, b1, w2, b2) in enumerate(params):
        partials = sc_agg(h, src, dst)
        b1r = b1.reshape(1, h_dim)
        b2r = b2.reshape(1, h_dim)
        if i < 2:
            h = mlp(h, partials, w1, b1r, w2, b2r)
        else:
            h = mlp_pool(h, partials, w1, b1r, w2, b2r, batch2d)
    return h


# speedup vs baseline: 10.0324x; 2.8697x over previous
import functools

import jax
import jax.numpy as jnp
import numpy as np
from jax import lax
from jax.experimental import pallas as pl
from jax.experimental.pallas import tpu as pltpu
from jax.experimental.pallas import tpu_sc as plsc

NC, NS = 2, 16
NW = NC * NS
CHUNK = 128


def _make_sc_agg(n_acc, n_chunks, d):
    rpt = n_acc // NS
    n_half = n_chunks // 2
    n_pairs = n_half // 2

    mesh = plsc.VectorSubcoreMesh(core_axis_name="c", subcore_axis_name="s")

    @functools.partial(
        pl.kernel,
        out_type=jax.ShapeDtypeStruct((NC, n_acc, d), jnp.float32),
        mesh=mesh,
        scratch_types=[
            pltpu.VMEM((n_half, CHUNK), jnp.int32),
            pltpu.VMEM((n_half, CHUNK), jnp.int32),
            pltpu.VMEM((CHUNK, d), jnp.float32),
            pltpu.VMEM((CHUNK, d), jnp.float32),
            pltpu.SemaphoreType.DMA,
            pltpu.SemaphoreType.DMA,
            pltpu.VMEM_SHARED((n_acc, d), jnp.float32),
        ],
    )
    def sc_agg(h_hbm, src_hbm, dst_hbm, out_hbm, src_v, dst_v, rows_a,
               rows_b, sem_a, sem_b, acc_sh):
        c = lax.axis_index("c")
        s = lax.axis_index("s")
        w = c * NS + s

        z16 = jnp.zeros((16,), jnp.float32)

        def zrow(r, carry):
            for c16 in range(d // 16):
                rows_a[r, pl.ds(c16 * 16, 16)] = z16
            return carry

        lax.fori_loop(0, CHUNK, zrow, 0)
        n_full = rpt // CHUNK
        for b in range(n_full):
            pltpu.sync_copy(
                rows_a, acc_sh.at[pl.ds(s * rpt + b * CHUNK, CHUNK)])
        rem = rpt - n_full * CHUNK
        if rem:
            pltpu.sync_copy(
                rows_a.at[pl.ds(0, rem)],
                acc_sh.at[pl.ds(s * rpt + n_full * CHUNK, rem)])
        plsc.subcore_barrier()

        for hf in range(2):
            pltpu.sync_copy(src_hbm.at[w].at[pl.ds(hf * n_half, n_half)],
                            src_v)
            pltpu.sync_copy(dst_hbm.at[w].at[pl.ds(hf * n_half, n_half)],
                            dst_v)
            pltpu.async_copy(h_hbm.at[src_v.at[0]], rows_a, sem_a)

            def body(p, carry):
                ja = 2 * p
                jb = 2 * p + 1
                pltpu.make_async_copy(
                    h_hbm.at[src_v.at[ja]], rows_a, sem_a).wait()
                pltpu.async_copy(h_hbm.at[src_v.at[jb]], rows_b, sem_b)
                pltpu.sync_copy(rows_a, acc_sh.at[dst_v.at[ja]], add=True)
                pltpu.make_async_copy(
                    h_hbm.at[src_v.at[jb]], rows_b, sem_b).wait()

                @pl.when(p < n_pairs - 1)
                def _():
                    pltpu.async_copy(
                        h_hbm.at[src_v.at[ja + 2]], rows_a, sem_a)

                pltpu.sync_copy(rows_b, acc_sh.at[dst_v.at[jb]], add=True)
                return carry

            lax.fori_loop(0, n_pairs, body, 0)
        plsc.subcore_barrier()

        pltpu.sync_copy(acc_sh.at[pl.ds(s * rpt, rpt)],
                        out_hbm.at[c].at[pl.ds(s * rpt, rpt)])

    return sc_agg


def _mlp_body(x_ref, p_ref, w1_ref, b1_ref, w2_ref, b2_ref, o_ref):
    n = x_ref.shape[0]
    m = x_ref[...] + p_ref[0, :n, :] + p_ref[1, :n, :]
    a = jnp.maximum(
        jnp.dot(m, w1_ref[...], preferred_element_type=jnp.float32)
        + b1_ref[...], 0.0)
    o_ref[...] = jnp.maximum(
        jnp.dot(a, w2_ref[...], preferred_element_type=jnp.float32)
        + b2_ref[...], 0.0)


def _mlp_pool_body(x_ref, p_ref, w1_ref, b1_ref, w2_ref, b2_ref, batch_ref,
                   o_ref):
    n = x_ref.shape[0]
    g = o_ref.shape[0]
    m = x_ref[...] + p_ref[0, :n, :] + p_ref[1, :n, :]
    a = jnp.maximum(
        jnp.dot(m, w1_ref[...], preferred_element_type=jnp.float32)
        + b1_ref[...], 0.0)
    h = jnp.maximum(
        jnp.dot(a, w2_ref[...], preferred_element_type=jnp.float32)
        + b2_ref[...], 0.0)
    gids = lax.broadcasted_iota(jnp.int32, (n, g), 1)
    oh = (batch_ref[...] == gids).astype(jnp.float32)
    sums = lax.dot_general(oh, h, (((0,), (0,)), ((), ())),
                           preferred_element_type=jnp.float32)
    counts = lax.dot_general(oh, jnp.ones((n, 1), jnp.float32),
                             (((0,), (0,)), ((), ())),
                             preferred_element_type=jnp.float32)
    o_ref[...] = sums / jnp.maximum(counts, 1.0)


def kernel(x, edge_index, batch, W1_0, b1_0, W2_0, b2_0, W1_1, b1_1, W2_1,
           b2_1, W1_2, b1_2, W2_2, b2_2):
    n_nodes, d = x.shape
    e = edge_index.shape[1]
    g = 64
    h_dim = W1_0.shape[1]

    n_acc = -(-(n_nodes + 1) // (8 * NS)) * (8 * NS)
    n_chunks = -(-e // (NW * CHUNK * 4)) * 4
    e_pad = NW * n_chunks * CHUNK

    pad = e_pad - e
    spare = n_acc - n_nodes
    pad_src = np.arange(pad, dtype=np.int32) * 37 % n_nodes
    pad_dst = n_nodes + np.arange(pad, dtype=np.int32) % spare
    src_flat = jnp.concatenate([edge_index[0], jnp.asarray(pad_src)])
    dst_flat = jnp.concatenate([edge_index[1], jnp.asarray(pad_dst)])
    src = src_flat.reshape(n_chunks * CHUNK, NW).T.reshape(
        NW, n_chunks, CHUNK)
    dst = dst_flat.reshape(n_chunks * CHUNK, NW).T.reshape(
        NW, n_chunks, CHUNK)

    sc_agg = _make_sc_agg(n_acc, n_chunks, d)

    mlp = pl.pallas_call(
        _mlp_body,
        out_shape=jax.ShapeDtypeStruct((n_nodes, h_dim), jnp.float32),
    )
    mlp_pool = pl.pallas_call(
        _mlp_pool_body,
        out_shape=jax.ShapeDtypeStruct((g, h_dim), jnp.float32),
    )

    params = [(W1_0, b1_0, W2_0, b2_0), (W1_1, b1_1, W2_1, b2_1),
              (W1_2, b1_2, W2_2, b2_2)]
    batch2d = batch.reshape(n_nodes, 1)

    h = x
    for i, (w1, b1, w2, b2) in enumerate(params):
        partials = sc_agg(h, src, dst)
        b1r = b1.reshape(1, h_dim)
        b2r = b2.reshape(1, h_dim)
        if i < 2:
            h = mlp(h, partials, w1, b1r, w2, b2r)
        else:
            h = mlp_pool(h, partials, w1, b1r, w2, b2r, batch2d)
    return h


# speedup vs baseline: 12.3479x; 1.2308x over previous
import functools

import jax
import jax.numpy as jnp
import numpy as np
from jax import lax
from jax.experimental import pallas as pl
from jax.experimental.pallas import tpu as pltpu
from jax.experimental.pallas import tpu_sc as plsc

NC, NS = 2, 16
NW = NC * NS
CHUNK = 64
NBUF = 4
NSTAGE = 4


def _make_sc_agg(n_acc, n_chunks, d):
    rpt = n_acc // NS
    n_stage = n_chunks // NSTAGE
    n_quads = n_stage // NBUF

    mesh = plsc.VectorSubcoreMesh(core_axis_name="c", subcore_axis_name="s")

    @functools.partial(
        pl.kernel,
        out_type=jax.ShapeDtypeStruct((NC, n_acc, d), jnp.float32),
        mesh=mesh,
        scratch_types=[
            pltpu.VMEM((n_stage, CHUNK), jnp.int32),
            pltpu.VMEM((n_stage, CHUNK), jnp.int32),
            [pltpu.VMEM((CHUNK, d), jnp.float32)] * NBUF,
            [pltpu.SemaphoreType.DMA] * NBUF,
            pltpu.VMEM_SHARED((n_acc, d), jnp.float32),
        ],
    )
    def sc_agg(h_hbm, src_hbm, dst_hbm, out_hbm, src_v, dst_v, rows,
               sems, acc_sh):
        c = lax.axis_index("c")
        s = lax.axis_index("s")
        w = c * NS + s

        z16 = jnp.zeros((16,), jnp.float32)

        def zrow(r, carry):
            for c16 in range(d // 16):
                rows[0][r, pl.ds(c16 * 16, 16)] = z16
            return carry

        lax.fori_loop(0, CHUNK, zrow, 0)
        n_full = rpt // CHUNK
        for b in range(n_full):
            pltpu.sync_copy(
                rows[0], acc_sh.at[pl.ds(s * rpt + b * CHUNK, CHUNK)])
        rem = rpt - n_full * CHUNK
        if rem:
            pltpu.sync_copy(
                rows[0].at[pl.ds(0, rem)],
                acc_sh.at[pl.ds(s * rpt + n_full * CHUNK, rem)])
        plsc.subcore_barrier()

        for hf in range(NSTAGE):
            pltpu.sync_copy(src_hbm.at[w].at[pl.ds(hf * n_stage, n_stage)],
                            src_v)
            pltpu.sync_copy(dst_hbm.at[w].at[pl.ds(hf * n_stage, n_stage)],
                            dst_v)
            for k in range(NBUF - 1):
                pltpu.async_copy(h_hbm.at[src_v.at[k]], rows[k], sems[k])

            def body(p, carry):
                base = NBUF * p
                for k in range(NBUF):
                    j = base + k
                    pltpu.make_async_copy(
                        h_hbm.at[src_v.at[j]], rows[k], sems[k]).wait()
                    kn = (k + NBUF - 1) % NBUF

                    @pl.when(j + NBUF - 1 < n_stage)
                    def _():
                        pltpu.async_copy(
                            h_hbm.at[src_v.at[j + NBUF - 1]], rows[kn],
                            sems[kn])

                    pltpu.sync_copy(rows[k], acc_sh.at[dst_v.at[j]],
                                    add=True)
                return carry

            lax.fori_loop(0, n_quads, body, 0)
        plsc.subcore_barrier()

        pltpu.sync_copy(acc_sh.at[pl.ds(s * rpt, rpt)],
                        out_hbm.at[c].at[pl.ds(s * rpt, rpt)])

    return sc_agg


def _mlp_body(x_ref, p_ref, w1_ref, b1_ref, w2_ref, b2_ref, o_ref):
    n = x_ref.shape[0]
    m = x_ref[...] + p_ref[0, :n, :] + p_ref[1, :n, :]
    a = jnp.maximum(
        jnp.dot(m, w1_ref[...], preferred_element_type=jnp.float32)
        + b1_ref[...], 0.0)
    o_ref[...] = jnp.maximum(
        jnp.dot(a, w2_ref[...], preferred_element_type=jnp.float32)
        + b2_ref[...], 0.0)


def _mlp_pool_body(x_ref, p_ref, w1_ref, b1_ref, w2_ref, b2_ref, batch_ref,
                   o_ref):
    n = x_ref.shape[0]
    g = o_ref.shape[0]
    m = x_ref[...] + p_ref[0, :n, :] + p_ref[1, :n, :]
    a = jnp.maximum(
        jnp.dot(m, w1_ref[...], preferred_element_type=jnp.float32)
        + b1_ref[...], 0.0)
    h = jnp.maximum(
        jnp.dot(a, w2_ref[...], preferred_element_type=jnp.float32)
        + b2_ref[...], 0.0)
    gids = lax.broadcasted_iota(jnp.int32, (n, g), 1)
    oh = (batch_ref[...] == gids).astype(jnp.float32)
    sums = lax.dot_general(oh, h, (((0,), (0,)), ((), ())),
                           preferred_element_type=jnp.float32)
    counts = lax.dot_general(oh, jnp.ones((n, 1), jnp.float32),
                             (((0,), (0,)), ((), ())),
                             preferred_element_type=jnp.float32)
    o_ref[...] = sums / jnp.maximum(counts, 1.0)


def kernel(x, edge_index, batch, W1_0, b1_0, W2_0, b2_0, W1_1, b1_1, W2_1,
           b2_1, W1_2, b1_2, W2_2, b2_2):
    n_nodes, d = x.shape
    e = edge_index.shape[1]
    g = 64
    h_dim = W1_0.shape[1]

    n_acc = -(-(n_nodes + 1) // (8 * NS)) * (8 * NS)
    nmul = NSTAGE * NBUF
    n_chunks = -(-e // (NW * CHUNK * nmul)) * nmul
    e_pad = NW * n_chunks * CHUNK

    pad = e_pad - e
    spare = n_acc - n_nodes
    pad_src = np.arange(pad, dtype=np.int32) * 37 % n_nodes
    pad_dst = n_nodes + np.arange(pad, dtype=np.int32) % spare
    src_flat = jnp.concatenate([edge_index[0], jnp.asarray(pad_src)])
    dst_flat = jnp.concatenate([edge_index[1], jnp.asarray(pad_dst)])
    src = src_flat.reshape(n_chunks * CHUNK, NW).T.reshape(
        NW, n_chunks, CHUNK)
    dst = dst_flat.reshape(n_chunks * CHUNK, NW).T.reshape(
        NW, n_chunks, CHUNK)

    sc_agg = _make_sc_agg(n_acc, n_chunks, d)

    mlp = pl.pallas_call(
        _mlp_body,
        out_shape=jax.ShapeDtypeStruct((n_nodes, h_dim), jnp.float32),
    )
    mlp_pool = pl.pallas_call(
        _mlp_pool_body,
        out_shape=jax.ShapeDtypeStruct((g, h_dim), jnp.float32),
    )

    params = [(W1_0, b1_0, W2_0, b2_0), (W1_1, b1_1, W2_1, b2_1),
              (W1_2, b1_2, W2_2, b2_2)]
    batch2d = batch.reshape(n_nodes, 1)

    h = x
    for i, (w1, b1, w2, b2) in enumerate(params):
        partials = sc_agg(h, src, dst)
        b1r = b1.reshape(1, h_dim)
        b2r = b2.reshape(1, h_dim)
        if i < 2:
            h = mlp(h, partials, w1, b1r, w2, b2r)
        else:
            h = mlp_pool(h, partials, w1, b1r, w2, b2r, batch2d)
    return h
